# Initial kernel scaffold; baseline (speedup 1.0000x reference)
import jax
import jax.numpy as jnp
from jax.experimental import pallas as pl


def kernel(x, edge_index, edge_attr, We, be, W1, b1, W2, b2):
    raise NotImplementedError("write your pallas kernel here")


# speedup vs baseline: 2.5235x; 2.5235x over previous
import functools

import jax
import jax.numpy as jnp
from jax import lax
from jax.experimental import pallas as pl
from jax.experimental.pallas import tpu as pltpu
from jax.experimental.pallas import tpu_sc as plsc

_NC = 2
_NS = 16
_LANES = 16


def _eproj_body(a_ref, w_ref, b_ref, o_ref):
    o_ref[...] = (
        jnp.dot(a_ref[...], w_ref[...], preferred_element_type=jnp.float32)
        + b_ref[...]
    )


def _edge_projection(edge_attr, We, be):
    E, DE = edge_attr.shape
    D = We.shape[1]
    BE = 3200
    assert E % BE == 0
    return pl.pallas_call(
        _eproj_body,
        grid=(E // BE,),
        in_specs=[
            pl.BlockSpec((BE, DE), lambda i: (i, 0)),
            pl.BlockSpec((DE, D), lambda i: (0, 0)),
            pl.BlockSpec((1, D), lambda i: (0, 0)),
        ],
        out_specs=pl.BlockSpec((BE, D), lambda i: (i, 0)),
        out_shape=jax.ShapeDtypeStruct((E, D), jnp.float32),
    )(edge_attr, We, be.reshape(1, D))


def _sc_edge_body(N, D, E, C, x_hbm, src_hbm, dst_hbm, e_hbm, out_hbm,
                  src_v, dst_v, xbuf, ebuf, zbuf, aggr_sh, sem):
    cid = lax.axis_index("c")
    sid = lax.axis_index("s")
    wid = cid * _NS + sid

    epw = E // (_NC * _NS)
    nchunk = epw // C
    zrows = zbuf.shape[0]
    nrow_chunks = N // zrows

    zvec = jnp.zeros((_LANES,), jnp.float32)

    def _zero_row(i, _):
        for j in range(D // _LANES):
            zbuf[i, pl.ds(j * _LANES, _LANES)] = zvec
        return 0

    lax.fori_loop(0, zrows, _zero_row, 0)
    for k in range((nrow_chunks + _NS - 1) // _NS):
        c = sid + k * _NS

        @pl.when(c < nrow_chunks)
        def _():
            pltpu.sync_copy(zbuf, aggr_sh.at[pl.ds(pl.multiple_of(c * zrows, 8), zrows)])

    plsc.subcore_barrier()

    def _chunk(j, _):
        ebase = pl.multiple_of(wid * epw + j * C, 8)
        pltpu.sync_copy(src_hbm.at[pl.ds(ebase, C)], src_v)
        pltpu.sync_copy(dst_hbm.at[pl.ds(ebase, C)], dst_v)
        pltpu.async_copy(x_hbm.at[src_v], xbuf, sem).wait()
        pltpu.sync_copy(e_hbm.at[pl.ds(ebase, C)], ebuf)

        def _row(i, _):
            for jj in range(D // _LANES):
                sl = pl.ds(jj * _LANES, _LANES)
                ebuf[i, sl] = jnp.maximum(xbuf[i, sl] + ebuf[i, sl], 0.0)
            return 0

        lax.fori_loop(0, C, _row, 0)
        pltpu.sync_copy(ebuf, aggr_sh.at[dst_v], add=True)
        return 0

    lax.fori_loop(0, nchunk, _chunk, 0)
    plsc.subcore_barrier()

    for k in range((nrow_chunks + _NS - 1) // _NS):
        c = sid + k * _NS

        @pl.when(c < nrow_chunks)
        def _():
            base = pl.multiple_of(c * zrows, 8)
            pltpu.sync_copy(aggr_sh.at[pl.ds(base, zrows)], zbuf)
            pltpu.sync_copy(zbuf, out_hbm.at[cid, pl.ds(base, zrows)])


def _sc_aggregate(x, edge_index, e):
    N, D = x.shape
    E = edge_index.shape[1]
    C = 80
    assert E % (_NC * _NS) == 0 and (E // (_NC * _NS)) % C == 0
    zrows = 80
    assert N % zrows == 0
    mesh = plsc.VectorSubcoreMesh(core_axis_name="c", subcore_axis_name="s")
    kern = functools.partial(
        pl.kernel,
        mesh=mesh,
        out_type=jax.ShapeDtypeStruct((_NC, N, D), jnp.float32),
        scratch_types=[
            pltpu.VMEM((C,), jnp.int32),
            pltpu.VMEM((C,), jnp.int32),
            pltpu.VMEM((C, D), jnp.float32),
            pltpu.VMEM((C, D), jnp.float32),
            pltpu.VMEM((zrows, D), jnp.float32),
            pltpu.VMEM_SHARED((N, D), jnp.float32),
            pltpu.SemaphoreType.DMA,
        ],
    )(functools.partial(_sc_edge_body, N, D, E, C))
    return kern(x, edge_index[0], edge_index[1], e)


def _mlp_body(x_ref, p_ref, w1_ref, b1_ref, w2_ref, b2_ref, o_ref):
    h = x_ref[...] + p_ref[0] + p_ref[1]
    h = jnp.maximum(
        jnp.dot(h, w1_ref[...], preferred_element_type=jnp.float32) + b1_ref[...],
        0.0,
    )
    o_ref[...] = (
        jnp.dot(h, w2_ref[...], preferred_element_type=jnp.float32) + b2_ref[...]
    )


def _node_mlp(x, partials, W1, b1, W2, b2):
    N, D = x.shape
    BN = 2000
    assert N % BN == 0
    return pl.pallas_call(
        _mlp_body,
        grid=(N // BN,),
        in_specs=[
            pl.BlockSpec((BN, D), lambda i: (i, 0)),
            pl.BlockSpec((_NC, BN, D), lambda i: (0, i, 0)),
            pl.BlockSpec((D, D), lambda i: (0, 0)),
            pl.BlockSpec((1, D), lambda i: (0, 0)),
            pl.BlockSpec((D, D), lambda i: (0, 0)),
            pl.BlockSpec((1, D), lambda i: (0, 0)),
        ],
        out_specs=pl.BlockSpec((BN, D), lambda i: (i, 0)),
        out_shape=jax.ShapeDtypeStruct((N, D), jnp.float32),
    )(x, partials, W1, b1.reshape(1, D), W2, b2.reshape(1, D))


def kernel(x, edge_index, edge_attr, We, be, W1, b1, W2, b2):
    e = _edge_projection(edge_attr, We, be)
    partials = _sc_aggregate(x, edge_index, e)
    return _node_mlp(x, partials, W1, b1, W2, b2)


# speedup vs baseline: 2.6743x; 1.0598x over previous
import functools

import jax
import jax.numpy as jnp
from jax import lax
from jax.experimental import pallas as pl
from jax.experimental.pallas import tpu as pltpu
from jax.experimental.pallas import tpu_sc as plsc

_NC = 2
_NS = 16
_LANES = 16


def _eproj_body(a_ref, w_ref, b_ref, o_ref):
    o_ref[...] = (
        jnp.dot(a_ref[...], w_ref[...], preferred_element_type=jnp.float32)
        + b_ref[...]
    )


def _edge_projection(edge_attr, We, be):
    E, DE = edge_attr.shape
    D = We.shape[1]
    BE = 3200
    assert E % BE == 0
    return pl.pallas_call(
        _eproj_body,
        grid=(E // BE,),
        in_specs=[
            pl.BlockSpec((BE, DE), lambda i: (i, 0)),
            pl.BlockSpec((DE, D), lambda i: (0, 0)),
            pl.BlockSpec((1, D), lambda i: (0, 0)),
        ],
        out_specs=pl.BlockSpec((BE, D), lambda i: (i, 0)),
        out_shape=jax.ShapeDtypeStruct((E, D), jnp.float32),
    )(edge_attr, We, be.reshape(1, D))


def _sc_edge_body(N, D, E, C, x_hbm, src_hbm, dst_hbm, e_hbm, out_hbm,
                  src_v, dst_v, xbuf, ebuf, zbuf, aggr_sh, sem):
    cid = lax.axis_index("c")
    sid = lax.axis_index("s")
    wid = cid * _NS + sid

    epw = E // (_NC * _NS)
    nchunk = epw // C
    zrows = zbuf.shape[0]
    nrow_chunks = N // zrows

    zvec = jnp.zeros((_LANES,), jnp.float32)

    def _zero_row(i, _):
        for j in range(D // _LANES):
            zbuf[i, pl.ds(j * _LANES, _LANES)] = zvec
        return 0

    lax.fori_loop(0, zrows, _zero_row, 0)
    for k in range((nrow_chunks + _NS - 1) // _NS):
        c = sid + k * _NS

        @pl.when(c < nrow_chunks)
        def _():
            pltpu.sync_copy(zbuf, aggr_sh.at[pl.ds(pl.multiple_of(c * zrows, 8), zrows)])

    plsc.subcore_barrier()

    def _chunk(j, _):
        ebase = pl.multiple_of(wid * epw + j * C, 8)
        pltpu.sync_copy(src_hbm.at[pl.ds(ebase, C)], src_v)
        pltpu.sync_copy(dst_hbm.at[pl.ds(ebase, C)], dst_v)
        pltpu.sync_copy(e_hbm.at[pl.ds(ebase, C)], ebuf)
        pltpu.async_copy(x_hbm.at[src_v], ebuf, sem, add=True).wait()

        def _row(i, _):
            for jj in range(D // _LANES):
                sl = pl.ds(jj * _LANES, _LANES)
                ebuf[i, sl] = jnp.maximum(ebuf[i, sl], 0.0)
            return 0

        lax.fori_loop(0, C, _row, 0)
        pltpu.sync_copy(ebuf, aggr_sh.at[dst_v], add=True)
        return 0

    lax.fori_loop(0, nchunk, _chunk, 0)
    plsc.subcore_barrier()

    for k in range((nrow_chunks + _NS - 1) // _NS):
        c = sid + k * _NS

        @pl.when(c < nrow_chunks)
        def _():
            base = pl.multiple_of(c * zrows, 8)
            pltpu.sync_copy(aggr_sh.at[pl.ds(base, zrows)], zbuf)
            pltpu.sync_copy(zbuf, out_hbm.at[cid, pl.ds(base, zrows)])


def _sc_aggregate(x, edge_index, e):
    N, D = x.shape
    E = edge_index.shape[1]
    C = 80
    assert E % (_NC * _NS) == 0 and (E // (_NC * _NS)) % C == 0
    zrows = 80
    assert N % zrows == 0
    mesh = plsc.VectorSubcoreMesh(core_axis_name="c", subcore_axis_name="s")
    kern = functools.partial(
        pl.kernel,
        mesh=mesh,
        out_type=jax.ShapeDtypeStruct((_NC, N, D), jnp.float32),
        scratch_types=[
            pltpu.VMEM((C,), jnp.int32),
            pltpu.VMEM((C,), jnp.int32),
            pltpu.VMEM((C, D), jnp.float32),
            pltpu.VMEM((C, D), jnp.float32),
            pltpu.VMEM((zrows, D), jnp.float32),
            pltpu.VMEM_SHARED((N, D), jnp.float32),
            pltpu.SemaphoreType.DMA,
        ],
    )(functools.partial(_sc_edge_body, N, D, E, C))
    return kern(x, edge_index[0], edge_index[1], e)


def _mlp_body(x_ref, p_ref, w1_ref, b1_ref, w2_ref, b2_ref, o_ref):
    h = x_ref[...] + p_ref[0] + p_ref[1]
    h = jnp.maximum(
        jnp.dot(h, w1_ref[...], preferred_element_type=jnp.float32) + b1_ref[...],
        0.0,
    )
    o_ref[...] = (
        jnp.dot(h, w2_ref[...], preferred_element_type=jnp.float32) + b2_ref[...]
    )


def _node_mlp(x, partials, W1, b1, W2, b2):
    N, D = x.shape
    BN = 2000
    assert N % BN == 0
    return pl.pallas_call(
        _mlp_body,
        grid=(N // BN,),
        in_specs=[
            pl.BlockSpec((BN, D), lambda i: (i, 0)),
            pl.BlockSpec((_NC, BN, D), lambda i: (0, i, 0)),
            pl.BlockSpec((D, D), lambda i: (0, 0)),
            pl.BlockSpec((1, D), lambda i: (0, 0)),
            pl.BlockSpec((D, D), lambda i: (0, 0)),
            pl.BlockSpec((1, D), lambda i: (0, 0)),
        ],
        out_specs=pl.BlockSpec((BN, D), lambda i: (i, 0)),
        out_shape=jax.ShapeDtypeStruct((N, D), jnp.float32),
    )(x, partials, W1, b1.reshape(1, D), W2, b2.reshape(1, D))


def kernel(x, edge_index, edge_attr, We, be, W1, b1, W2, b2):
    e = _edge_projection(edge_attr, We, be)
    partials = _sc_aggregate(x, edge_index, e)
    return _node_mlp(x, partials, W1, b1, W2, b2)


# speedup vs baseline: 3.6126x; 1.3508x over previous
import functools

import jax
import jax.numpy as jnp
from jax import lax
from jax.experimental import pallas as pl
from jax.experimental.pallas import tpu as pltpu
from jax.experimental.pallas import tpu_sc as plsc

_NC = 2
_NS = 16
_LANES = 16


def _eproj_body(a_ref, w_ref, b_ref, o_ref):
    o_ref[...] = (
        jnp.dot(a_ref[...], w_ref[...], preferred_element_type=jnp.float32)
        + b_ref[...]
    )


def _edge_projection(edge_attr, We, be):
    E, DE = edge_attr.shape
    D = We.shape[1]
    BE = 3200
    assert E % BE == 0
    return pl.pallas_call(
        _eproj_body,
        grid=(E // BE,),
        in_specs=[
            pl.BlockSpec((BE, DE), lambda i: (i, 0)),
            pl.BlockSpec((DE, D), lambda i: (0, 0)),
            pl.BlockSpec((1, D), lambda i: (0, 0)),
        ],
        out_specs=pl.BlockSpec((BE, D), lambda i: (i, 0)),
        out_shape=jax.ShapeDtypeStruct((E, D), jnp.float32),
    )(edge_attr, We, be.reshape(1, D))


def _sc_edge_body(N, D, E, C, x_hbm, src_hbm, dst_hbm, e_hbm, out_hbm,
                  src_v0, dst_v0, src_v1, dst_v1, ebuf0, ebuf1, zbuf, aggr_sh,
                  sem_i0, sem_i1, sem_e0, sem_e1, sem_g0, sem_g1):
    cid = lax.axis_index("c")
    sid = lax.axis_index("s")
    wid = cid * _NS + sid

    epw = E // (_NC * _NS)
    nchunk = epw // C
    zrows = zbuf.shape[0]
    nrow_chunks = N // zrows

    zvec = jnp.zeros((_LANES,), jnp.float32)

    def _zero_row(i, _):
        for j in range(D // _LANES):
            zbuf[i, pl.ds(j * _LANES, _LANES)] = zvec
        return 0

    lax.fori_loop(0, zrows, _zero_row, 0)
    for k in range((nrow_chunks + _NS - 1) // _NS):
        c = sid + k * _NS

        @pl.when(c < nrow_chunks)
        def _():
            pltpu.sync_copy(zbuf, aggr_sh.at[pl.ds(pl.multiple_of(c * zrows, 8), zrows)])

    plsc.subcore_barrier()

    def _base(j):
        return pl.multiple_of(wid * epw + j * C, 8)

    def issue_idx(j, sv, dv, sem):
        pltpu.async_copy(src_hbm.at[pl.ds(_base(j), C)], sv, sem)
        pltpu.async_copy(dst_hbm.at[pl.ds(_base(j), C)], dv, sem)

    def wait_idx(j, sv, dv, sem):
        pltpu.make_async_copy(src_hbm.at[pl.ds(_base(j), C)], sv, sem).wait()
        pltpu.make_async_copy(dst_hbm.at[pl.ds(_base(j), C)], dv, sem).wait()

    def issue_e(j, eb, sem):
        pltpu.async_copy(e_hbm.at[pl.ds(_base(j), C)], eb, sem)

    def wait_e(j, eb, sem):
        pltpu.make_async_copy(e_hbm.at[pl.ds(_base(j), C)], eb, sem).wait()

    def issue_g(sv, eb, sem):
        pltpu.async_copy(x_hbm.at[sv], eb, sem, add=True)

    def wait_g(sv, eb, sem):
        pltpu.make_async_copy(x_hbm.at[sv], eb, sem).wait()

    def process(eb, dv):
        def _row(i, _):
            for jj in range(D // _LANES):
                sl = pl.ds(jj * _LANES, _LANES)
                eb[i, sl] = jnp.maximum(eb[i, sl], 0.0)
            return 0

        lax.fori_loop(0, C, _row, 0)
        pltpu.sync_copy(eb, aggr_sh.at[dv], add=True)

    assert nchunk % 2 == 1 and nchunk >= 3
    npair = nchunk // 2

    issue_idx(0, src_v0, dst_v0, sem_i0)
    issue_idx(1, src_v1, dst_v1, sem_i1)
    issue_e(0, ebuf0, sem_e0)
    issue_e(1, ebuf1, sem_e1)
    wait_idx(0, src_v0, dst_v0, sem_i0)
    wait_e(0, ebuf0, sem_e0)
    issue_g(src_v0, ebuf0, sem_g0)

    def _pair(t, _):
        j0 = 2 * t
        j1 = j0 + 1
        wait_g(src_v0, ebuf0, sem_g0)
        process(ebuf0, dst_v0)
        issue_e(j0 + 2, ebuf0, sem_e0)
        wait_idx(j1, src_v1, dst_v1, sem_i1)
        wait_e(j1, ebuf1, sem_e1)
        issue_g(src_v1, ebuf1, sem_g1)
        issue_idx(j0 + 2, src_v0, dst_v0, sem_i0)
        wait_g(src_v1, ebuf1, sem_g1)
        process(ebuf1, dst_v1)

        @pl.when(j1 + 2 < nchunk)
        def _():
            issue_e(j1 + 2, ebuf1, sem_e1)

        wait_idx(j0 + 2, src_v0, dst_v0, sem_i0)
        wait_e(j0 + 2, ebuf0, sem_e0)
        issue_g(src_v0, ebuf0, sem_g0)

        @pl.when(j1 + 2 < nchunk)
        def _():
            issue_idx(j1 + 2, src_v1, dst_v1, sem_i1)

        return 0

    lax.fori_loop(0, npair, _pair, 0)
    wait_g(src_v0, ebuf0, sem_g0)
    process(ebuf0, dst_v0)
    plsc.subcore_barrier()

    for k in range((nrow_chunks + _NS - 1) // _NS):
        c = sid + k * _NS

        @pl.when(c < nrow_chunks)
        def _():
            base = pl.multiple_of(c * zrows, 8)
            pltpu.sync_copy(aggr_sh.at[pl.ds(base, zrows)], zbuf)
            pltpu.sync_copy(zbuf, out_hbm.at[cid, pl.ds(base, zrows)])


def _sc_aggregate(x, edge_index, e):
    N, D = x.shape
    E = edge_index.shape[1]
    C = 80
    assert E % (_NC * _NS) == 0 and (E // (_NC * _NS)) % C == 0
    zrows = 80
    assert N % zrows == 0
    mesh = plsc.VectorSubcoreMesh(core_axis_name="c", subcore_axis_name="s")
    kern = functools.partial(
        pl.kernel,
        mesh=mesh,
        out_type=jax.ShapeDtypeStruct((_NC, N, D), jnp.float32),
        scratch_types=[
            pltpu.VMEM((C,), jnp.int32),
            pltpu.VMEM((C,), jnp.int32),
            pltpu.VMEM((C,), jnp.int32),
            pltpu.VMEM((C,), jnp.int32),
            pltpu.VMEM((C, D), jnp.float32),
            pltpu.VMEM((C, D), jnp.float32),
            pltpu.VMEM((zrows, D), jnp.float32),
            pltpu.VMEM_SHARED((N, D), jnp.float32),
            pltpu.SemaphoreType.DMA,
            pltpu.SemaphoreType.DMA,
            pltpu.SemaphoreType.DMA,
            pltpu.SemaphoreType.DMA,
            pltpu.SemaphoreType.DMA,
            pltpu.SemaphoreType.DMA,
        ],
    )(functools.partial(_sc_edge_body, N, D, E, C))
    return kern(x, edge_index[0], edge_index[1], e)


def _mlp_body(x_ref, p_ref, w1_ref, b1_ref, w2_ref, b2_ref, o_ref):
    h = x_ref[...] + p_ref[0] + p_ref[1]
    h = jnp.maximum(
        jnp.dot(h, w1_ref[...], preferred_element_type=jnp.float32) + b1_ref[...],
        0.0,
    )
    o_ref[...] = (
        jnp.dot(h, w2_ref[...], preferred_element_type=jnp.float32) + b2_ref[...]
    )


def _node_mlp(x, partials, W1, b1, W2, b2):
    N, D = x.shape
    BN = 2000
    assert N % BN == 0
    return pl.pallas_call(
        _mlp_body,
        grid=(N // BN,),
        in_specs=[
            pl.BlockSpec((BN, D), lambda i: (i, 0)),
            pl.BlockSpec((_NC, BN, D), lambda i: (0, i, 0)),
            pl.BlockSpec((D, D), lambda i: (0, 0)),
            pl.BlockSpec((1, D), lambda i: (0, 0)),
            pl.BlockSpec((D, D), lambda i: (0, 0)),
            pl.BlockSpec((1, D), lambda i: (0, 0)),
        ],
        out_specs=pl.BlockSpec((BN, D), lambda i: (i, 0)),
        out_shape=jax.ShapeDtypeStruct((N, D), jnp.float32),
    )(x, partials, W1, b1.reshape(1, D), W2, b2.reshape(1, D))


def kernel(x, edge_index, edge_attr, We, be, W1, b1, W2, b2):
    e = _edge_projection(edge_attr, We, be)
    partials = _sc_aggregate(x, edge_index, e)
    return _node_mlp(x, partials, W1, b1, W2, b2)


# speedup vs baseline: 4.4478x; 1.2312x over previous
import functools

import jax
import jax.numpy as jnp
from jax import lax
from jax.experimental import pallas as pl
from jax.experimental.pallas import tpu as pltpu
from jax.experimental.pallas import tpu_sc as plsc

_NC = 2
_NS = 16
_LANES = 16


def _eproj_body(a_ref, w_ref, b_ref, o_ref):
    o_ref[...] = (
        jnp.dot(a_ref[...], w_ref[...], preferred_element_type=jnp.float32)
        + b_ref[...]
    )


def _edge_projection(edge_attr, We, be):
    E, DE = edge_attr.shape
    D = We.shape[1]
    BE = 3200
    assert E % BE == 0
    return pl.pallas_call(
        _eproj_body,
        grid=(E // BE,),
        in_specs=[
            pl.BlockSpec((BE, DE), lambda i: (i, 0)),
            pl.BlockSpec((DE, D), lambda i: (0, 0)),
            pl.BlockSpec((1, D), lambda i: (0, 0)),
        ],
        out_specs=pl.BlockSpec((BE, D), lambda i: (i, 0)),
        out_shape=jax.ShapeDtypeStruct((E, D), jnp.float32),
    )(edge_attr, We, be.reshape(1, D))


_NB = 4


def _sc_edge_body(N, D, E, C, x_hbm, src_hbm, dst_hbm, e_hbm, out_hbm, *sc):
    src_v = sc[0:_NB]
    dst_v = sc[_NB:2 * _NB]
    ebuf = sc[2 * _NB:3 * _NB]
    zbuf = sc[3 * _NB]
    aggr_sh = sc[3 * _NB + 1]
    sem_i = sc[3 * _NB + 2:3 * _NB + 2 + _NB]
    sem_e = sc[3 * _NB + 2 + _NB:3 * _NB + 2 + 2 * _NB]
    sem_g = sc[3 * _NB + 2 + 2 * _NB:3 * _NB + 2 + 3 * _NB]
    sem_s = sc[3 * _NB + 2 + 3 * _NB:3 * _NB + 2 + 4 * _NB]
    cid = lax.axis_index("c")
    sid = lax.axis_index("s")
    wid = cid * _NS + sid

    epw = E // (_NC * _NS)
    nchunk = epw // C
    zrows = zbuf.shape[0]
    nrow_chunks = N // zrows

    zvec = jnp.zeros((_LANES,), jnp.float32)

    def _zero_row(i, _):
        for j in range(D // _LANES):
            zbuf[i, pl.ds(j * _LANES, _LANES)] = zvec
        return 0

    lax.fori_loop(0, zrows, _zero_row, 0)
    for k in range((nrow_chunks + _NS - 1) // _NS):
        c = sid + k * _NS

        @pl.when(c < nrow_chunks)
        def _():
            pltpu.sync_copy(zbuf, aggr_sh.at[pl.ds(pl.multiple_of(c * zrows, 8), zrows)])

    plsc.subcore_barrier()

    def _base(j):
        return pl.multiple_of(wid * epw + j * C, 8)

    def issue_idx(j, b):
        pltpu.async_copy(src_hbm.at[pl.ds(_base(j), C)], src_v[b], sem_i[b])
        pltpu.async_copy(dst_hbm.at[pl.ds(_base(j), C)], dst_v[b], sem_i[b])

    def wait_idx(j, b):
        pltpu.make_async_copy(src_hbm.at[pl.ds(_base(j), C)], src_v[b], sem_i[b]).wait()
        pltpu.make_async_copy(dst_hbm.at[pl.ds(_base(j), C)], dst_v[b], sem_i[b]).wait()

    def issue_e(j, b):
        pltpu.async_copy(e_hbm.at[pl.ds(_base(j), C)], ebuf[b], sem_e[b])

    def wait_e(j, b):
        pltpu.make_async_copy(e_hbm.at[pl.ds(_base(j), C)], ebuf[b], sem_e[b]).wait()

    def issue_g(b):
        pltpu.async_copy(x_hbm.at[src_v[b]], ebuf[b], sem_g[b], add=True)

    def wait_g(b):
        pltpu.make_async_copy(x_hbm.at[src_v[b]], ebuf[b], sem_g[b]).wait()

    def issue_s(b):
        pltpu.async_copy(ebuf[b], aggr_sh.at[dst_v[b]], sem_s[b], add=True)

    def wait_s(b):
        pltpu.make_async_copy(ebuf[b], aggr_sh.at[dst_v[b]], sem_s[b]).wait()

    def relu(b):
        eb = ebuf[b]

        @plsc.parallel_loop(0, C, step=1, unroll=4)
        def _row(i):
            for jj in range(D // _LANES):
                sl = pl.ds(jj * _LANES, _LANES)
                eb[i, sl] = jnp.maximum(eb[i, sl], 0.0)

    assert nchunk % _NB == 1 and nchunk >= _NB + 1
    nquad = nchunk // _NB

    for b in range(_NB - 1):
        issue_idx(b, b)
        issue_e(b, b)
    wait_idx(0, 0)
    wait_e(0, 0)
    issue_g(0)

    def _step(j, b):
        wait_g(b)
        b1 = (b + 1) % _NB
        wait_idx(j + 1, b1)
        wait_e(j + 1, b1)
        issue_g(b1)
        relu(b)
        issue_s(b)
        b3 = (b + _NB - 1) % _NB

        @pl.when(j > 0)
        def _():
            wait_s(b3)

        @pl.when(j + _NB - 1 < nchunk)
        def _():
            issue_e(j + _NB - 1, b3)
            issue_idx(j + _NB - 1, b3)

    def _quad(t, _):
        for k in range(_NB):
            _step(_NB * t + k, k)
        return 0

    lax.fori_loop(0, nquad, _quad, 0)
    wait_g(0)
    relu(0)
    issue_s(0)
    wait_s(_NB - 1)
    wait_s(0)
    plsc.subcore_barrier()

    for k in range((nrow_chunks + _NS - 1) // _NS):
        c = sid + k * _NS

        @pl.when(c < nrow_chunks)
        def _():
            base = pl.multiple_of(c * zrows, 8)
            pltpu.sync_copy(aggr_sh.at[pl.ds(base, zrows)], zbuf)
            pltpu.sync_copy(zbuf, out_hbm.at[cid, pl.ds(base, zrows)])


def _sc_aggregate(x, edge_index, e):
    N, D = x.shape
    E = edge_index.shape[1]
    C = 80
    assert E % (_NC * _NS) == 0 and (E // (_NC * _NS)) % C == 0
    zrows = 40
    assert N % zrows == 0
    mesh = plsc.VectorSubcoreMesh(core_axis_name="c", subcore_axis_name="s")
    kern = functools.partial(
        pl.kernel,
        mesh=mesh,
        out_type=jax.ShapeDtypeStruct((_NC, N, D), jnp.float32),
        scratch_types=(
            [pltpu.VMEM((C,), jnp.int32) for _ in range(2 * _NB)]
            + [pltpu.VMEM((C, D), jnp.float32) for _ in range(_NB)]
            + [
                pltpu.VMEM((zrows, D), jnp.float32),
                pltpu.VMEM_SHARED((N, D), jnp.float32),
            ]
            + [pltpu.SemaphoreType.DMA for _ in range(4 * _NB)]
        ),
    )(functools.partial(_sc_edge_body, N, D, E, C))
    return kern(x, edge_index[0], edge_index[1], e)


def _mlp_body(x_ref, p_ref, w1_ref, b1_ref, w2_ref, b2_ref, o_ref):
    h = x_ref[...] + p_ref[0] + p_ref[1]
    h = jnp.maximum(
        jnp.dot(h, w1_ref[...], preferred_element_type=jnp.float32) + b1_ref[...],
        0.0,
    )
    o_ref[...] = (
        jnp.dot(h, w2_ref[...], preferred_element_type=jnp.float32) + b2_ref[...]
    )


def _node_mlp(x, partials, W1, b1, W2, b2):
    N, D = x.shape
    BN = 2000
    assert N % BN == 0
    return pl.pallas_call(
        _mlp_body,
        grid=(N // BN,),
        in_specs=[
            pl.BlockSpec((BN, D), lambda i: (i, 0)),
            pl.BlockSpec((_NC, BN, D), lambda i: (0, i, 0)),
            pl.BlockSpec((D, D), lambda i: (0, 0)),
            pl.BlockSpec((1, D), lambda i: (0, 0)),
            pl.BlockSpec((D, D), lambda i: (0, 0)),
            pl.BlockSpec((1, D), lambda i: (0, 0)),
        ],
        out_specs=pl.BlockSpec((BN, D), lambda i: (i, 0)),
        out_shape=jax.ShapeDtypeStruct((N, D), jnp.float32),
    )(x, partials, W1, b1.reshape(1, D), W2, b2.reshape(1, D))


def kernel(x, edge_index, edge_attr, We, be, W1, b1, W2, b2):
    e = _edge_projection(edge_attr, We, be)
    partials = _sc_aggregate(x, edge_index, e)
    return _node_mlp(x, partials, W1, b1, W2, b2)
